# baseline (speedup 1.0000x reference)
import jax
import jax.numpy as jnp
from jax import lax
from jax.experimental import pallas as pl
from jax.experimental.pallas import tpu as pltpu
from jax.experimental.pallas import tpu_sc as plsc

_E = 6400000
_K = 128
_LANES = 16
_CH = 4000
_NBUF = 7
_UNROLL = 5


def _make_kernel(n_elems, k_dim, num_cores, num_subcores, chunk, nbuf,
                 interpret=False):
    n_workers = num_cores * num_subcores
    per_tile = n_elems // n_workers
    assert per_tile * n_workers == n_elems
    n_chunks = per_tile // chunk
    assert n_chunks * chunk == per_tile
    n_vecs = chunk // _LANES
    assert n_vecs * _LANES == chunk

    mesh = plsc.VectorSubcoreMesh(
        core_axis_name="c",
        subcore_axis_name="s",
        num_cores=num_cores,
        num_subcores=num_subcores,
    )

    def body(param_hbm, i1_hbm, i2_hbm, out_hbm, tbl, *rest):
        i1v = rest[0:nbuf]
        i2v = rest[nbuf:2 * nbuf]
        ov = rest[2 * nbuf:3 * nbuf]
        s1 = rest[3 * nbuf:4 * nbuf]
        s2 = rest[4 * nbuf:5 * nbuf]
        so = rest[5 * nbuf:6 * nbuf]

        wid = lax.axis_index("s") * num_cores + lax.axis_index("c")
        base = wid * per_tile
        pltpu.sync_copy(param_hbm, tbl)

        def in_start(g, b):
            off = base + g * chunk
            pltpu.async_copy(i1_hbm.at[pl.ds(off, chunk)], i1v[b], s1[b])
            pltpu.async_copy(i2_hbm.at[pl.ds(off, chunk)], i2v[b], s2[b])

        def in_wait(b):
            pltpu.make_async_copy(i1_hbm.at[pl.ds(base, chunk)], i1v[b], s1[b]).wait()
            pltpu.make_async_copy(i2_hbm.at[pl.ds(base, chunk)], i2v[b], s2[b]).wait()

        def compute(b):
            @plsc.parallel_loop(0, n_vecs, 1, unroll=_UNROLL)
            def _(i):
                a = i1v[b][pl.ds(i * _LANES, _LANES)]
                c = i2v[b][pl.ds(i * _LANES, _LANES)]
                row = jnp.minimum(a, c)
                col = jnp.maximum(a, c)
                lin = row * k_dim + col
                g16 = plsc.load_gather(tbl, [lin])
                ov[b][pl.ds(i * _LANES, _LANES)] = jnp.maximum(g16, 0.0)

        def out_start(g, b):
            off = base + g * chunk
            pltpu.async_copy(ov[b], out_hbm.at[pl.ds(off, chunk)], so[b])

        def out_wait(b):
            pltpu.make_async_copy(ov[b], out_hbm.at[pl.ds(base, chunk)], so[b]).wait()

        for g in range(min(nbuf, n_chunks)):
            in_start(g, g)

        for g in range(n_chunks):
            b = g % nbuf
            in_wait(b)
            if g >= nbuf:
                out_wait(b)
            compute(b)
            out_start(g, b)
            if g + nbuf < n_chunks:
                in_start(g + nbuf, b)

        for b in range(min(nbuf, n_chunks)):
            out_wait(b)

    return pl.kernel(
        body,
        out_type=jax.ShapeDtypeStruct((n_elems,), jnp.float32),
        mesh=mesh,
        scratch_types=(
            [pltpu.VMEM((k_dim * k_dim,), jnp.float32)]
            + [pltpu.VMEM((chunk,), jnp.int32) for _ in range(nbuf)]
            + [pltpu.VMEM((chunk,), jnp.int32) for _ in range(nbuf)]
            + [pltpu.VMEM((chunk,), jnp.float32) for _ in range(nbuf)]
            + [pltpu.SemaphoreType.DMA for _ in range(3 * nbuf)]
        ),
        compiler_params=pltpu.CompilerParams(needs_layout_passes=False),
        interpret=interpret,
    )


@jax.jit
def kernel(param, index1, index2):
    info = plsc.get_sparse_core_info()
    f = _make_kernel(_E, _K, info.num_cores, info.num_subcores, _CH, _NBUF)
    return f(param.reshape(-1), index1, index2)


# speedup vs baseline: 1.1525x; 1.1525x over previous
import jax
import jax.numpy as jnp
from jax import lax
from jax.experimental import pallas as pl
from jax.experimental.pallas import tpu as pltpu
from jax.experimental.pallas import tpu_sc as plsc

_E = 6400000
_K = 128
_LANES = 16
_CH = 4000
_NBUF = 5
_UNROLL = 5


def _make_kernel(n_elems, k_dim, num_cores, num_subcores, chunk, nbuf,
                 interpret=False):
    n_workers = num_cores * num_subcores
    per_tile = n_elems // n_workers
    assert per_tile * n_workers == n_elems
    n_chunks = per_tile // chunk
    assert n_chunks * chunk == per_tile and n_chunks % nbuf == 0
    n_vecs = chunk // _LANES
    assert n_vecs * _LANES == chunk
    n_groups = n_chunks // nbuf

    mesh = plsc.VectorSubcoreMesh(
        core_axis_name="c",
        subcore_axis_name="s",
        num_cores=num_cores,
        num_subcores=num_subcores,
    )

    def body(param_hbm, i1_hbm, i2_hbm, out_hbm, tbl, *rest):
        i1v = rest[0:nbuf]
        i2v = rest[nbuf:2 * nbuf]
        ov = rest[2 * nbuf:3 * nbuf]
        s1 = rest[3 * nbuf:4 * nbuf]
        s2 = rest[4 * nbuf:5 * nbuf]
        so = rest[5 * nbuf:6 * nbuf]

        wid = lax.axis_index("s") * num_cores + lax.axis_index("c")
        base = wid * per_tile
        pltpu.sync_copy(param_hbm, tbl)

        def in_start(g, b):
            off = base + g * chunk
            pltpu.async_copy(i1_hbm.at[pl.ds(off, chunk)], i1v[b], s1[b])
            pltpu.async_copy(i2_hbm.at[pl.ds(off, chunk)], i2v[b], s2[b])

        def in_wait(b):
            pltpu.make_async_copy(i1_hbm.at[pl.ds(base, chunk)], i1v[b], s1[b]).wait()
            pltpu.make_async_copy(i2_hbm.at[pl.ds(base, chunk)], i2v[b], s2[b]).wait()

        def compute(b):
            @plsc.parallel_loop(0, n_vecs, 1, unroll=_UNROLL)
            def _(i):
                a = i1v[b][pl.ds(i * _LANES, _LANES)]
                c = i2v[b][pl.ds(i * _LANES, _LANES)]
                row = jnp.minimum(a, c)
                col = jnp.maximum(a, c)
                lin = row * k_dim + col
                g16 = plsc.load_gather(tbl, [lin])
                ov[b][pl.ds(i * _LANES, _LANES)] = jnp.maximum(g16, 0.0)

        def out_start(g, b):
            off = base + g * chunk
            pltpu.async_copy(ov[b], out_hbm.at[pl.ds(off, chunk)], so[b])

        def out_wait(b):
            pltpu.make_async_copy(ov[b], out_hbm.at[pl.ds(base, chunk)], so[b]).wait()

        for b in range(nbuf):
            in_start(b, b)

        def group(gi, carry):
            g0 = gi * nbuf
            for b in range(nbuf):
                in_wait(b)

                @pl.when(gi > 0)
                def _():
                    out_wait(b)

                compute(b)
                out_start(g0 + b, b)

                @pl.when(gi < n_groups - 1)
                def _():
                    in_start(g0 + b + nbuf, b)

            return carry

        lax.fori_loop(0, n_groups, group, 0)
        for b in range(nbuf):
            out_wait(b)

    return pl.kernel(
        body,
        out_type=jax.ShapeDtypeStruct((n_elems,), jnp.float32),
        mesh=mesh,
        scratch_types=(
            [pltpu.VMEM((k_dim * k_dim,), jnp.float32)]
            + [pltpu.VMEM((chunk,), jnp.int32) for _ in range(nbuf)]
            + [pltpu.VMEM((chunk,), jnp.int32) for _ in range(nbuf)]
            + [pltpu.VMEM((chunk,), jnp.float32) for _ in range(nbuf)]
            + [pltpu.SemaphoreType.DMA for _ in range(3 * nbuf)]
        ),
        compiler_params=pltpu.CompilerParams(needs_layout_passes=False),
        interpret=interpret,
    )


@jax.jit
def kernel(param, index1, index2):
    info = plsc.get_sparse_core_info()
    f = _make_kernel(_E, _K, info.num_cores, info.num_subcores, _CH, _NBUF)
    return f(param.reshape(-1), index1, index2)
